# baseline (speedup 1.0000x reference)
import functools

import jax
import jax.numpy as jnp
from jax import lax
from jax.experimental import pallas as pl
from jax.experimental.pallas import tpu as pltpu
from jax.experimental.pallas import tpu_sc as plsc

_N_SEQ = 4
_SEQ_LEN = 1000
_NF = 5143
_NI = 300


def kernel(flame_books, inner_books, idx_to_sequence, sequence, frame):
    flame4 = flame_books.reshape(_N_SEQ, _SEQ_LEN, _NF, 3)
    inner4 = inner_books.reshape(_N_SEQ, _SEQ_LEN, _NI, 3)
    seq1 = jnp.asarray(sequence, jnp.int32).reshape(1)
    frame1 = jnp.asarray(frame, jnp.int32).reshape(1)
    idxmap = idx_to_sequence.astype(jnp.int32)

    mesh = plsc.VectorSubcoreMesh(core_axis_name="c", subcore_axis_name="s")

    @functools.partial(
        pl.kernel,
        mesh=mesh,
        out_type=jax.ShapeDtypeStruct((_NF + _NI, 3), jnp.float32),
        compiler_params=pltpu.CompilerParams(
            needs_layout_passes=False, use_tc_tiling_on_sc=False),
        scratch_types=[
            pltpu.VMEM((16,), jnp.int32),
            pltpu.VMEM((16,), jnp.int32),
            pltpu.VMEM((_NF + _NI, 3), jnp.float32),
        ],
    )
    def body(flame_hbm, inner_hbm, map_hbm, seq_hbm, frame_hbm, out_hbm,
             sf_v, map_v, buf_v):
        wid = lax.axis_index("s") * 2 + lax.axis_index("c")

        @pl.when(wid == 0)
        def _():
            pltpu.sync_copy(seq_hbm, sf_v.at[pl.ds(0, 1)])
            pltpu.sync_copy(frame_hbm, sf_v.at[pl.ds(8, 1)])
            pltpu.sync_copy(map_hbm, map_v.at[pl.ds(0, _N_SEQ)])
            sf = sf_v[...]
            s_bcast = jnp.full((16,), sf[0], jnp.int32)
            s = plsc.load_gather(map_v, [s_bcast])[0]
            f = sf[8]
            pltpu.sync_copy(flame_hbm.at[s, f], buf_v.at[pl.ds(0, _NF)])
            pltpu.sync_copy(inner_hbm.at[s, f], buf_v.at[pl.ds(_NF, _NI)])
            pltpu.sync_copy(buf_v, out_hbm)

    return body(flame4, inner4, idxmap, seq1, frame1)
